# baseline (speedup 1.0000x reference)
import functools

import jax
import jax.numpy as jnp
from jax import lax
from jax.experimental import pallas as pl
from jax.experimental.pallas import tpu as pltpu
from jax.experimental.pallas import tpu_sc as plsc

VOCAB = 1000000
DIM = 64
NDOCS = 16384
L = 200
LP = 256

NC = 2
NS = 16
NW = NC * NS
DPW = NDOCS // NW
CH = 16
NCHUNK = DPW // CH

VB = 25600
GRID = 40
VPAD = VB * GRID
KB = DIM * 128
QB = VB // 128
EB = VB * DIM // 128

DB = 2048


def _matvec_body(emb_ref, bd_ref, out_ref):
    a = emb_ref[...].reshape(QB, KB)
    out_ref[...] = jax.lax.dot_general(
        a, bd_ref[...], (((1,), (0,)), ((), ())),
        preferred_element_type=jnp.float32)


def _scores_tc(emb_flat2d, bdiag):
    out2d = pl.pallas_call(
        _matvec_body,
        grid=(GRID,),
        in_specs=[
            pl.BlockSpec((EB, 128), lambda i: (i, 0)),
            pl.BlockSpec((KB, 128), lambda i: (0, 0)),
        ],
        out_specs=pl.BlockSpec((QB, 128), lambda i: (i, 0)),
        out_shape=jax.ShapeDtypeStruct((VPAD // 128, 128), jnp.float32),
    )(emb_flat2d, bdiag)
    return out2d.reshape(-1)


def _pad_body(docs_ref, out_ref):
    out_ref[:, 0:L] = docs_ref[...]
    out_ref[:, L:LP] = jnp.zeros((DB, LP - L), jnp.int32)


def _pad_docs_tc(docs):
    padded = pl.pallas_call(
        _pad_body,
        grid=(NDOCS // DB,),
        in_specs=[pl.BlockSpec((DB, L), lambda i: (i, 0))],
        out_specs=pl.BlockSpec((DB, LP), lambda i: (i, 0)),
        out_shape=jax.ShapeDtypeStruct((NDOCS, LP), jnp.int32),
    )(docs)
    return padded.reshape(-1)


def _pool_sc(scores, docs1d, bias16):
    mesh = plsc.VectorSubcoreMesh(core_axis_name="c", subcore_axis_name="s")

    @functools.partial(
        pl.kernel,
        mesh=mesh,
        out_type=jax.ShapeDtypeStruct((NDOCS,), jnp.float32),
        scratch_types=[
            pltpu.VMEM((LP * CH,), jnp.int32),
            pltpu.VMEM((LP * CH,), jnp.float32),
            pltpu.VMEM((DPW,), jnp.float32),
            pltpu.VMEM((16,), jnp.float32),
            pltpu.SemaphoreType.DMA,
        ],
        compiler_params=pltpu.CompilerParams(needs_layout_passes=False),
    )
    def k(scores_hbm, docs_hbm, bias_hbm, out_hbm, idx_v, vals_v, res_v,
          bias_v, sem):
        wid = lax.axis_index("s") * NC + lax.axis_index("c")
        pltpu.sync_copy(bias_hbm, bias_v)
        bvec = bias_v[...]
        lane_doc = lax.iota(jnp.int32, 16) * LP

        def chunk_body(ci, _):
            flat = (wid * NCHUNK + ci) * (LP * CH)
            pltpu.sync_copy(docs_hbm.at[pl.ds(flat, LP * CH)], idx_v)
            pltpu.async_copy(scores_hbm.at[idx_v], vals_v, sem).wait()

            def red(l, acc):
                return acc + plsc.load_gather(vals_v, [lane_doc + l])

            acc = lax.fori_loop(0, L, red, jnp.zeros((CH,), jnp.float32))
            x = acc * (1.0 / L) + bvec
            res_v[pl.ds(ci * CH, CH)] = 1.0 / (1.0 + jnp.exp(-x))
            return 0

        lax.fori_loop(0, NCHUNK, chunk_body, 0)
        pltpu.sync_copy(res_v, out_hbm.at[pl.ds(wid * DPW, DPW)])

    return k(scores, docs1d, bias16)


def kernel(docs, thetas, emb, W, b):
    del thetas
    emb_flat2d = emb.reshape(-1, 128)
    bdiag = jnp.kron(jnp.eye(128, dtype=jnp.float32), W.astype(jnp.float32))
    bias16 = jnp.broadcast_to(b, (16,)).astype(jnp.float32)
    scores = _scores_tc(emb_flat2d, bdiag)
    docs1d = _pad_docs_tc(docs)
    return _pool_sc(scores, docs1d, bias16)
